# baseline (speedup 1.0000x reference)
import jax
import jax.numpy as jnp
from jax import lax
from jax.experimental import pallas as pl
from jax.experimental.pallas import tpu as pltpu
from jax.experimental.pallas import tpu_sc as plsc

STU_NUM = 100000
PP_DIM = 128
BATCH = 16384
X_DIM = 128
OUT_DIM = PP_DIM + X_DIM

NC = 2
NS = 16
NW = NC * NS
B_PER_W = BATCH // NW

C = 128
NCHUNK = B_PER_W // C
NBUF = 4

XCHUNK = 8
NXBUF = 4
XROWS = BATCH // NC // XCHUNK


def _gather_concat(x_hbm, idx_hbm, table_hbm, out_hbm,
                   idx_v, rows_v, spx, isem, gsem, rsem, xisem, xosem):
    cid = lax.axis_index("c")
    sid = lax.axis_index("s")
    wid = sid * NC + cid
    base = wid * B_PER_W

    def x_in(k):
        row0 = (cid * XCHUNK + k) * XROWS
        return pltpu.make_async_copy(
            x_hbm.at[pl.ds(row0, XROWS)], spx.at[k % NXBUF],
            xisem.at[k % NXBUF])

    def x_out(k):
        row0 = (cid * XCHUNK + k) * XROWS
        return pltpu.make_async_copy(
            spx.at[k % NXBUF],
            out_hbm.at[pl.ds(row0, XROWS), pl.ds(PP_DIM, X_DIM)],
            xosem.at[k % NXBUF])

    @pl.when(sid == 0)
    def _():
        for k in range(NXBUF):
            x_in(k).start()

    for k in range(NCHUNK):
        pltpu.async_copy(idx_hbm.at[pl.ds(base + k * C, C)], idx_v.at[k],
                         isem)
    for k in range(NCHUNK):
        pltpu.make_async_copy(idx_hbm.at[pl.ds(base + k * C, C)],
                              idx_v.at[k], isem).wait()

    def gather(k, b):
        return pltpu.make_async_copy(table_hbm.at[idx_v.at[k]],
                                     rows_v.at[b], gsem.at[b])

    def rows_write(k, b):
        return pltpu.make_async_copy(
            rows_v.at[b],
            out_hbm.at[pl.ds(base + k * C, C), pl.ds(0, PP_DIM)],
            rsem.at[b])

    for k in range(min(NBUF, NCHUNK)):
        gather(k, k % NBUF).start()

    @pl.when(sid == 0)
    def _():
        for k in range(XCHUNK):
            x_in(k).wait()
            x_out(k).start()
            if k + NXBUF < XCHUNK:
                x_out(k).wait()
                x_in(k + NXBUF).start()

    for k in range(NCHUNK):
        b = k % NBUF
        gather(k, b).wait()
        rows_write(k, b).start()
        nk = k + NBUF
        if nk < NCHUNK:
            rows_write(k, b).wait()
            gather(nk, b).start()
    for k in range(max(0, NCHUNK - NBUF), NCHUNK):
        rows_write(k, k % NBUF).wait()

    @pl.when(sid == 0)
    def _():
        for k in range(max(0, XCHUNK - NXBUF), XCHUNK):
            x_out(k).wait()


@jax.jit
def _run(x, stu_id, ability_emb):
    mesh = plsc.VectorSubcoreMesh(core_axis_name="c", subcore_axis_name="s")
    return pl.kernel(
        _gather_concat,
        out_type=jax.ShapeDtypeStruct((BATCH, OUT_DIM), jnp.float32),
        mesh=mesh,
        scratch_types=[
            pltpu.VMEM((NCHUNK, C), jnp.int32),
            pltpu.VMEM((NBUF, C, PP_DIM), jnp.float32),
            pltpu.VMEM_SHARED((NXBUF, XROWS, X_DIM), jnp.float32),
            pltpu.SemaphoreType.DMA,
            pltpu.SemaphoreType.DMA((NBUF,)),
            pltpu.SemaphoreType.DMA((NBUF,)),
            pltpu.SemaphoreType.DMA((NXBUF,)),
            pltpu.SemaphoreType.DMA((NXBUF,)),
        ],
    )(x, stu_id, ability_emb)


def kernel(x, stu_id, ability_emb):
    return _run(x, stu_id.astype(jnp.int32), ability_emb)
